# baseline (speedup 1.0000x reference)
import functools

import jax
import jax.numpy as jnp
from jax import lax
from jax.experimental import pallas as pl
from jax.experimental.pallas import tpu as pltpu
from jax.experimental.pallas import tpu_sc as plsc

BATCH = 4096
N_SENT = 50
EMB = 32
B = BATCH * N_SENT
NC = 2
NS = 16
NW = NC * NS
BPW = B // NW
RPD = 128
K = BPW // RPD
CHUNK = 10
NCHUNK = K // CHUNK
ROWS_PER_CHUNK = CHUNK * RPD

_mesh = plsc.VectorSubcoreMesh(core_axis_name="c", subcore_axis_name="s")


@functools.partial(
    pl.kernel,
    out_type=jax.ShapeDtypeStruct((B, EMB), jnp.float32),
    mesh=_mesh,
    scratch_types=[
        pltpu.VMEM((K, RPD), jnp.int32),
        pltpu.VMEM((ROWS_PER_CHUNK, EMB), jnp.float32),
        pltpu.SemaphoreType.DMA,
    ],
    compiler_params=pltpu.CompilerParams(use_tc_tiling_on_sc=False),
)
def _gather(table_hbm, idx_hbm, out_hbm, idx_v, rows_v, sem):
    wid = lax.axis_index("s") * NC + lax.axis_index("c")
    base = wid * BPW
    pltpu.sync_copy(idx_hbm.at[wid], idx_v)

    @pl.loop(0, NCHUNK)
    def _chunk(c):
        copies = [
            pltpu.async_copy(
                table_hbm.at[idx_v.at[c * CHUNK + j]],
                rows_v.at[pl.ds(j * RPD, RPD)],
                sem,
            )
            for j in range(CHUNK)
        ]
        for cp in copies:
            cp.wait()
        pltpu.sync_copy(
            rows_v, out_hbm.at[pl.ds(base + c * ROWS_PER_CHUNK, ROWS_PER_CHUNK)]
        )


def kernel(sentences, sent_emb_table):
    idx = sentences.reshape(NW, K, RPD)
    out = _gather(sent_emb_table, idx)
    return out.reshape(BATCH, N_SENT, EMB)


# speedup vs baseline: 1.0033x; 1.0033x over previous
import functools

import jax
import jax.numpy as jnp
from jax import lax
from jax.experimental import pallas as pl
from jax.experimental.pallas import tpu as pltpu
from jax.experimental.pallas import tpu_sc as plsc

BATCH = 4096
N_SENT = 50
EMB = 32
B = BATCH * N_SENT
NC = 2
NS = 16
NW = NC * NS
BPW = B // NW
RPD = 128
K = BPW // RPD
CHUNK = 5
NCHUNK = K // CHUNK
ROWS_PER_CHUNK = CHUNK * RPD

_mesh = plsc.VectorSubcoreMesh(core_axis_name="c", subcore_axis_name="s")


@functools.partial(
    pl.kernel,
    out_type=jax.ShapeDtypeStruct((B, EMB), jnp.float32),
    mesh=_mesh,
    scratch_types=[
        pltpu.VMEM((K, RPD), jnp.int32),
        pltpu.VMEM((ROWS_PER_CHUNK, EMB), jnp.float32),
        pltpu.VMEM((ROWS_PER_CHUNK, EMB), jnp.float32),
        pltpu.SemaphoreType.DMA,
        pltpu.SemaphoreType.DMA,
    ],
    compiler_params=pltpu.CompilerParams(use_tc_tiling_on_sc=False),
)
def _gather(table_hbm, idx_hbm, out_hbm, idx_v, buf_a, buf_b, gsem, wsem):
    wid = lax.axis_index("s") * NC + lax.axis_index("c")
    base = wid * BPW
    pltpu.sync_copy(idx_hbm.at[wid], idx_v)

    def fire(c, buf):
        for j in range(CHUNK):
            pltpu.async_copy(
                table_hbm.at[idx_v.at[c * CHUNK + j]],
                buf.at[pl.ds(j * RPD, RPD)],
                gsem,
            )

    def drain_gather(buf):
        for j in range(CHUNK):
            pltpu.make_async_copy(
                table_hbm.at[pl.ds(0, RPD)], buf.at[pl.ds(0, RPD)], gsem
            ).wait()

    def writeback(c, buf):
        pltpu.async_copy(
            buf, out_hbm.at[pl.ds(base + c * ROWS_PER_CHUNK, ROWS_PER_CHUNK)], wsem
        )

    def drain_writeback(buf):
        pltpu.make_async_copy(
            buf, out_hbm.at[pl.ds(base, ROWS_PER_CHUNK)], wsem
        ).wait()

    fire(0, buf_a)
    fire(1, buf_b)
    drain_gather(buf_a)
    writeback(0, buf_a)

    @pl.loop(1, NCHUNK // 2)
    def _pair(i):
        a = 2 * i
        drain_writeback(buf_a)
        fire(a, buf_a)
        drain_gather(buf_b)
        writeback(a - 1, buf_b)
        drain_writeback(buf_b)
        fire(a + 1, buf_b)
        drain_gather(buf_a)
        writeback(a, buf_a)

    drain_gather(buf_b)
    writeback(NCHUNK - 1, buf_b)
    drain_writeback(buf_a)
    drain_writeback(buf_b)


def kernel(sentences, sent_emb_table):
    idx = sentences.reshape(NW, K, RPD)
    out = _gather(sent_emb_table, idx)
    return out.reshape(BATCH, N_SENT, EMB)


# speedup vs baseline: 1.1592x; 1.1554x over previous
import functools

import jax
import jax.numpy as jnp
from jax import lax
from jax.experimental import pallas as pl
from jax.experimental.pallas import tpu as pltpu
from jax.experimental.pallas import tpu_sc as plsc

BATCH = 4096
N_SENT = 50
EMB = 32
VOCAB = 1000000
SUPER = 128
NSUP = VOCAB * EMB // SUPER
NC = 2
NS = 16
NW = NC * NS
RPD = 128
NJ = BATCH // RPD
NBLK = N_SENT * NJ
BPW = NBLK // NW

_mesh = plsc.VectorSubcoreMesh(core_axis_name="c", subcore_axis_name="s")


@functools.partial(
    pl.kernel,
    out_type=jax.ShapeDtypeStruct((N_SENT, EMB // 8, NJ, 8, RPD), jnp.float32),
    mesh=_mesh,
    scratch_types=[
        pltpu.VMEM((BPW, RPD), jnp.int32),
        pltpu.VMEM((RPD,), jnp.int32),
        pltpu.VMEM((RPD,), jnp.int32),
        pltpu.VMEM((RPD, SUPER), jnp.float32),
        pltpu.VMEM((RPD, SUPER), jnp.float32),
        pltpu.VMEM((EMB // 8, 8, RPD), jnp.float32),
        pltpu.VMEM((EMB // 8, 8, RPD), jnp.float32),
        pltpu.SemaphoreType.DMA,
        pltpu.SemaphoreType.DMA,
    ],
    compiler_params=pltpu.CompilerParams(
        use_tc_tiling_on_sc=False, needs_layout_passes=False
    ),
)
def _gather(table_hbm, idx_hbm, out_hbm, idx_v, sup_a, sup_b,
            rows_a, rows_b, trans_a, trans_b, gsem, wsem):
    wid = lax.axis_index("s") * NC + lax.axis_index("c")
    base = wid * BPW
    pltpu.sync_copy(idx_hbm.at[pl.ds(base, BPW)], idx_v)

    lane = lax.iota(jnp.int32, 16)
    row_ids = [lane + (c16 * 16) for c16 in range(8)]

    def fire(g, sup, rows):
        for c16 in range(8):
            sl = pl.ds(c16 * 16, 16)
            sup[sl] = lax.shift_right_logical(idx_v[g, sl], 2)
        pltpu.async_copy(table_hbm.at[sup], rows, gsem)

    def drain_gather(rows):
        pltpu.make_async_copy(table_hbm.at[pl.ds(0, RPD)], rows, gsem).wait()

    def transpose(g, rows, trans):
        for c16 in range(8):
            colbase = (idx_v[g, pl.ds(c16 * 16, 16)] & 3) * EMB
            for f in range(EMB):
                fi, fr = divmod(f, 8)
                trans[fi, fr, pl.ds(c16 * 16, 16)] = plsc.load_gather(
                    rows, [row_ids[c16], colbase + f]
                )

    def writeback(g, trans):
        gid = base + g
        s = gid // NJ
        j = lax.rem(gid, NJ)
        for fi in range(EMB // 8):
            pltpu.async_copy(trans.at[fi], out_hbm.at[s, fi, j], wsem)

    def drain_write(trans):
        for fi in range(EMB // 8):
            pltpu.make_async_copy(trans.at[fi], out_hbm.at[0, fi, 0], wsem).wait()

    fire(0, sup_a, rows_a)
    fire(1, sup_b, rows_b)
    drain_gather(rows_a)
    transpose(0, rows_a, trans_a)
    writeback(0, trans_a)
    drain_gather(rows_b)
    transpose(1, rows_b, trans_b)
    writeback(1, trans_b)
    fire(2, sup_a, rows_a)
    fire(3, sup_b, rows_b)

    @pl.loop(1, BPW // 2 - 1)
    def _pair(i):
        a = 2 * i
        drain_gather(rows_a)
        drain_write(trans_a)
        transpose(a, rows_a, trans_a)
        writeback(a, trans_a)
        fire(a + 2, sup_a, rows_a)
        drain_gather(rows_b)
        drain_write(trans_b)
        transpose(a + 1, rows_b, trans_b)
        writeback(a + 1, trans_b)
        fire(a + 3, sup_b, rows_b)

    drain_gather(rows_a)
    drain_write(trans_a)
    transpose(BPW - 2, rows_a, trans_a)
    writeback(BPW - 2, trans_a)
    drain_gather(rows_b)
    drain_write(trans_b)
    transpose(BPW - 1, rows_b, trans_b)
    writeback(BPW - 1, trans_b)
    drain_write(trans_a)
    drain_write(trans_b)


def kernel(sentences, sent_emb_table):
    idx = sentences.T.reshape(NBLK, RPD)
    table = sent_emb_table.reshape(NSUP, SUPER)
    out5 = _gather(table, idx)
    return out5.transpose(2, 4, 0, 1, 3).reshape(BATCH, N_SENT, EMB)
